# baseline (speedup 1.0000x reference)
import functools

import jax
import jax.numpy as jnp
from jax import lax
from jax.experimental import pallas as pl
from jax.experimental.pallas import tpu as pltpu
from jax.experimental.pallas import tpu_sc as plsc

_E, _O, _D = 64, 192, 768
_ET = 16
_TT = 2048
_NE = _E // _ET

_NW = 32
_TW = 64
_L = 16
_KW = _TT // _TW


def _tc_body(aggw_ref, ew_ref, x_ref, gw_ref, eb_ref, ow_ref, ob_ref, ab_ref,
             lp_ref, v_scr, amat_scr):
    i = pl.program_id(0)

    @pl.when(i == 0)
    def _amat():
        a = aggw_ref[...]
        a_rep = jnp.concatenate([a] * _ET, axis=1)
        rows = jax.lax.broadcasted_iota(jnp.int32, (_ET, _ET * _O), 0)
        cols = jax.lax.broadcasted_iota(jnp.int32, (_ET, _ET * _O), 1)
        amat_scr[...] = jnp.where(rows == cols // _O,
                                  jnp.broadcast_to(a_rep, (_ET, _ET * _O)),
                                  0.0)

    @pl.when(i < _NE)
    def _vred():
        v_scr[pl.ds(i * _ET, _ET), :] = jax.lax.dot_general(
            amat_scr[...], ew_ref[...], (((1,), (0,)), ((), ())),
            preferred_element_type=jnp.float32)

    @pl.when(i >= _NE)
    def _tokens():
        x = x_ref[...]
        lgt = jax.lax.dot_general(
            gw_ref[...], x, (((1,), (1,)), ((), ())),
            preferred_element_type=jnp.float32)
        pt = jax.lax.dot_general(
            v_scr[...], x, (((1,), (1,)), ((), ())),
            preferred_element_type=jnp.float32)
        base = jax.lax.dot_general(
            ow_ref[...], x, (((1,), (1,)), ((), ())),
            preferred_element_type=jnp.float32)
        cvec = jnp.sum(eb_ref[...] * aggw_ref[...], axis=1,
                       keepdims=True)
        pt = pt + base + cvec + (ob_ref[0, 0] + ab_ref[0, 0])
        for k in range(_KW):
            lp_ref[k, 0:_E, :] = lgt[:, k * _TW:(k + 1) * _TW]
            lp_ref[k, _E:2 * _E, :] = pt[:, k * _TW:(k + 1) * _TW]


def _sc_body(lp_hbm, out_hbm, lp_v, o_v):
    wid = lax.axis_index("s") * 2 + lax.axis_index("c")
    pltpu.sync_copy(lp_hbm.at[wid], lp_v)
    for g in range(_TW // _L):
        sl = pl.ds(g * _L, _L)
        m = jnp.full((_L,), -jnp.inf, jnp.float32)
        val = jnp.zeros((_L,), jnp.float32)
        for j in range(_E):
            row = lp_v[j, sl]
            better = row > m
            m = jnp.where(better, row, m)
            val = jnp.where(better, lp_v[_E + j, sl], val)
        o_v[sl] = val
    pltpu.sync_copy(o_v, out_hbm.at[pl.ds(wid * _TW, _TW)])


def kernel(x, gate_w, expert_w, expert_b, agg_w, agg_b, orig_w, orig_b):
    B, S, D = x.shape
    G = agg_w.shape[0]
    T = B * S
    hs = x.reshape(T, D)
    ob = orig_b.reshape(1, 1)
    ab = agg_b.reshape(1, 1)
    nt = T // _TT
    last_e = _NE - 1

    lp = pl.pallas_call(
        _tc_body,
        grid=(_NE + nt,),
        in_specs=[
            pl.BlockSpec((1, _O), lambda i: (0, 0)),
            pl.BlockSpec((_ET * _O, _D),
                         lambda i: (jnp.minimum(i, last_e), 0)),
            pl.BlockSpec((_TT, _D),
                         lambda i: (jnp.maximum(i - _NE, 0), 0)),
            pl.BlockSpec((_E, _D), lambda i: (0, 0)),
            pl.BlockSpec((_E, _O), lambda i: (0, 0)),
            pl.BlockSpec((1, _D), lambda i: (0, 0)),
            pl.BlockSpec((1, 1), lambda i: (0, 0)),
            pl.BlockSpec((1, 1), lambda i: (0, 0)),
        ],
        out_specs=pl.BlockSpec((_KW, 2 * _E, _TW),
                               lambda i: (jnp.maximum(i - _NE, 0), 0, 0)),
        out_shape=jax.ShapeDtypeStruct((_NW, 2 * _E, _TW), jnp.float32),
        scratch_shapes=[pltpu.VMEM((_E, _D), jnp.float32),
                        pltpu.VMEM((_ET, _ET * _O), jnp.float32)],
    )(agg_w, expert_w.reshape(_E * _O, D), hs, gate_w, expert_b, orig_w,
      ob, ab)

    sc = functools.partial(
        pl.kernel,
        mesh=plsc.VectorSubcoreMesh(core_axis_name="c", subcore_axis_name="s"),
        out_type=jax.ShapeDtypeStruct((T,), jnp.float32),
        scratch_types=[pltpu.VMEM((2 * _E, _TW), jnp.float32),
                       pltpu.VMEM((_TW,), jnp.float32)],
    )(_sc_body)
    out = sc(lp)

    return out.reshape(B, S, G)


# speedup vs baseline: 1.0183x; 1.0183x over previous
import functools

import jax
import jax.numpy as jnp
from jax import lax
from jax.experimental import pallas as pl
from jax.experimental.pallas import tpu as pltpu
from jax.experimental.pallas import tpu_sc as plsc

_E, _O, _D = 64, 192, 768
_ET = 16
_TT = 1024
_NE = _E // _ET

_NW = 32
_TW = 64
_L = 16
_KW = _TT // _TW


def _tc_body(aggw_ref, ew_ref, x_ref, gw_ref, eb_ref, ow_ref, ob_ref, ab_ref,
             lp_ref, v_scr, amat_scr):
    i = pl.program_id(0)

    @pl.when(i == 0)
    def _amat():
        a = aggw_ref[...]
        a_rep = jnp.concatenate([a] * _ET, axis=1)
        rows = jax.lax.broadcasted_iota(jnp.int32, (_ET, _ET * _O), 0)
        cols = jax.lax.broadcasted_iota(jnp.int32, (_ET, _ET * _O), 1)
        amat_scr[...] = jnp.where(rows == cols // _O,
                                  jnp.broadcast_to(a_rep, (_ET, _ET * _O)),
                                  0.0)

    @pl.when(i < _NE)
    def _vred():
        v_scr[pl.ds(i * _ET, _ET), :] = jax.lax.dot_general(
            amat_scr[...], ew_ref[...], (((1,), (0,)), ((), ())),
            preferred_element_type=jnp.float32)

    @pl.when(i >= _NE)
    def _tokens():
        x = x_ref[...]
        lgt = jax.lax.dot_general(
            gw_ref[...], x, (((1,), (1,)), ((), ())),
            preferred_element_type=jnp.float32)
        pt = jax.lax.dot_general(
            v_scr[...], x, (((1,), (1,)), ((), ())),
            preferred_element_type=jnp.float32)
        base = jax.lax.dot_general(
            ow_ref[...], x, (((1,), (1,)), ((), ())),
            preferred_element_type=jnp.float32)
        cvec = jnp.sum(eb_ref[...] * aggw_ref[...], axis=1,
                       keepdims=True)
        pt = pt + base + cvec + (ob_ref[0, 0] + ab_ref[0, 0])
        for k in range(_KW):
            lp_ref[k, 0:_E, :] = lgt[:, k * _TW:(k + 1) * _TW]
            lp_ref[k, _E:2 * _E, :] = pt[:, k * _TW:(k + 1) * _TW]


def _sc_body(lp_hbm, out_hbm, lp_v, o_v):
    wid = lax.axis_index("s") * 2 + lax.axis_index("c")
    pltpu.sync_copy(lp_hbm.at[wid], lp_v)
    for g in range(_TW // _L):
        sl = pl.ds(g * _L, _L)
        m = jnp.full((_L,), -jnp.inf, jnp.float32)
        val = jnp.zeros((_L,), jnp.float32)
        for j in range(_E):
            row = lp_v[j, sl]
            better = row > m
            m = jnp.where(better, row, m)
            val = jnp.where(better, lp_v[_E + j, sl], val)
        o_v[sl] = val
    pltpu.sync_copy(o_v, out_hbm.at[pl.ds(wid * _TW, _TW)])


def kernel(x, gate_w, expert_w, expert_b, agg_w, agg_b, orig_w, orig_b):
    B, S, D = x.shape
    G = agg_w.shape[0]
    T = B * S
    hs = x.reshape(T, D)
    ob = orig_b.reshape(1, 1)
    ab = agg_b.reshape(1, 1)
    nt = T // _TT
    last_e = _NE - 1

    lp = pl.pallas_call(
        _tc_body,
        grid=(_NE + nt,),
        in_specs=[
            pl.BlockSpec((1, _O), lambda i: (0, 0)),
            pl.BlockSpec((_ET * _O, _D),
                         lambda i: (jnp.minimum(i, last_e), 0)),
            pl.BlockSpec((_TT, _D),
                         lambda i: (jnp.maximum(i - _NE, 0), 0)),
            pl.BlockSpec((_E, _D), lambda i: (0, 0)),
            pl.BlockSpec((_E, _O), lambda i: (0, 0)),
            pl.BlockSpec((1, _D), lambda i: (0, 0)),
            pl.BlockSpec((1, 1), lambda i: (0, 0)),
            pl.BlockSpec((1, 1), lambda i: (0, 0)),
        ],
        out_specs=pl.BlockSpec((_KW, 2 * _E, _TW),
                               lambda i: (jnp.maximum(i - _NE, 0), 0, 0)),
        out_shape=jax.ShapeDtypeStruct((_NW, 2 * _E, _TW), jnp.float32),
        scratch_shapes=[pltpu.VMEM((_E, _D), jnp.float32),
                        pltpu.VMEM((_ET, _ET * _O), jnp.float32)],
    )(agg_w, expert_w.reshape(_E * _O, D), hs, gate_w, expert_b, orig_w,
      ob, ab)

    sc = functools.partial(
        pl.kernel,
        mesh=plsc.VectorSubcoreMesh(core_axis_name="c", subcore_axis_name="s"),
        out_type=jax.ShapeDtypeStruct((T,), jnp.float32),
        scratch_types=[pltpu.VMEM((2 * _E, _TW), jnp.float32),
                       pltpu.VMEM((_TW,), jnp.float32)],
    )(_sc_body)
    out = sc(lp)

    return out.reshape(B, S, G)
